# Initial kernel scaffold; baseline (speedup 1.0000x reference)
import jax
import jax.numpy as jnp
from jax.experimental import pallas as pl
from jax.experimental.pallas import tpu as pltpu
from jax.experimental.pallas import tpu_sc as plsc

_WINDOW = 128


def kernel(x, table):
    batch, hist = x.shape
    _, dim = table.shape
    n = batch * hist
    idx = x.reshape(1, n).astype(jnp.int32)

    mesh = plsc.VectorSubcoreMesh(
        core_axis_name="core", subcore_axis_name="subcore"
    )

    @pl.kernel(
        out_type=jax.ShapeDtypeStruct((n, dim), table.dtype),
        mesh=mesh,
    )
    def gather_kernel(table_hbm, idx_hbm, out_hbm):
        def body(idx_vmem, out_vmem):
            pltpu.sync_copy(table_hbm.at[idx_vmem.at[0]], out_vmem)

        pltpu.emit_pipeline(
            body,
            grid=(n // _WINDOW,),
            in_specs=[
                pl.BlockSpec((1, _WINDOW), index_map=lambda i: (0, i))
            ],
            out_specs=[
                pl.BlockSpec((_WINDOW, dim), index_map=lambda i: (i, 0))
            ],
            core_axis_name=("core", "subcore"),
            dimension_semantics=(pltpu.PARALLEL,),
        )(idx_hbm, out_hbm)

    out = gather_kernel(table, idx)
    return out.reshape(batch, hist, dim)


# speedup vs baseline: 4.3249x; 4.3249x over previous
import jax
import jax.numpy as jnp
from jax import lax
from jax.experimental import pallas as pl
from jax.experimental.pallas import tpu as pltpu
from jax.experimental.pallas import tpu_sc as plsc

_NC = 2
_NS = 16
_NW = _NC * _NS
_ROWS_PER_BATCH = 50
_BATCH_CHUNK = 8
_CHUNK = _BATCH_CHUNK * _ROWS_PER_BATCH


def kernel(x, table):
    batch, hist = x.shape
    vocab, dim = table.shape
    n = batch * hist
    batches_per_w = batch // _NW
    idx = x.reshape(n).astype(jnp.int32)
    table128 = jnp.pad(table, ((0, 0), (0, 128 - dim)))

    mesh = plsc.VectorSubcoreMesh(core_axis_name="c", subcore_axis_name="s")

    @pl.kernel(
        out_type=jax.ShapeDtypeStruct((batch, hist, dim), table.dtype),
        mesh=mesh,
        scratch_types=[
            pltpu.VMEM((_CHUNK,), jnp.int32),
            pltpu.VMEM((_CHUNK, 128), table.dtype),
            pltpu.VMEM((_CHUNK, 64), table.dtype),
            pltpu.SemaphoreType.DMA,
        ],
    )
    def gather_kernel(table_hbm, idx_hbm, out_hbm, idx_v, rows_v, out_v, sem):
        wid = lax.axis_index("s") * _NC + lax.axis_index("c")
        batch0 = wid * batches_per_w

        @pl.loop(0, batches_per_w, step=_BATCH_CHUNK)
        def _(boff):
            base = (batch0 + boff) * hist
            pltpu.sync_copy(idx_hbm.at[pl.ds(base, _CHUNK)], idx_v)
            pltpu.async_copy(table_hbm.at[idx_v], rows_v, sem).wait()

            @pl.loop(0, _CHUNK)
            def _(r):
                for c in range(4):
                    slc = (pl.ds(r, 1), pl.ds(c * 16, 16))
                    out_v.at[slc][...] = rows_v.at[slc][...]

            for i in range(_BATCH_CHUNK):
                pltpu.sync_copy(
                    out_v.at[pl.ds(i * hist, hist)],
                    out_hbm.at[batch0 + boff + i],
                )

    return gather_kernel(table128, idx)


# speedup vs baseline: 5.0783x; 1.1742x over previous
import jax
import jax.numpy as jnp
from jax import lax
from jax.experimental import pallas as pl
from jax.experimental.pallas import tpu as pltpu
from jax.experimental.pallas import tpu_sc as plsc

_NC = 2
_NS = 16
_NW = _NC * _NS
_HIST = 50
_BCHUNK = 4
_W = _BCHUNK * _HIST
_NCHUNKS = 32
_PER_W = _W * _NCHUNKS


def kernel(x, table):
    batch, hist = x.shape
    vocab, dim = table.shape
    n = batch * hist
    idx = x.reshape(n).astype(jnp.int32)
    table128 = jnp.pad(table, ((0, 0), (0, 128 - dim)))

    mesh = plsc.VectorSubcoreMesh(core_axis_name="c", subcore_axis_name="s")

    @pl.kernel(
        out_type=jax.ShapeDtypeStruct((batch, hist, dim), table.dtype),
        mesh=mesh,
        scratch_types=[
            pltpu.VMEM((_PER_W,), jnp.int32),
            pltpu.VMEM((_W, 128), table.dtype),
            pltpu.VMEM((_W, 128), table.dtype),
            pltpu.VMEM((_BCHUNK, _HIST, dim), table.dtype),
            pltpu.VMEM((_BCHUNK, _HIST, dim), table.dtype),
            pltpu.SemaphoreType.DMA,
            pltpu.SemaphoreType.DMA,
        ],
    )
    def gather_kernel(table_hbm, idx_hbm, out_hbm, idx_v, rows0, rows1,
                      out0, out1, sem0, sem1):
        wid = lax.axis_index("s") * _NC + lax.axis_index("c")
        base = wid * _PER_W
        batch0 = wid * (_PER_W // hist)

        pltpu.sync_copy(idx_hbm.at[pl.ds(base, _PER_W)], idx_v)

        def start(g, rows, sem):
            pltpu.async_copy(
                table_hbm.at[idx_v.at[pl.ds(g * _W, _W)]], rows, sem
            )

        def wait(rows, sem):
            pltpu.make_async_copy(table_hbm.at[pl.ds(0, _W)], rows, sem).wait()

        def emit(g, rows, out_v):
            @pl.loop(0, _BCHUNK)
            def _(b):
                @pl.loop(0, _HIST)
                def _(r):
                    for c in range(dim // 16):
                        lanes = pl.ds(c * 16, 16)
                        out_v.at[b, pl.ds(r, 1), lanes][...] = (
                            rows.at[pl.ds(b * _HIST + r, 1), lanes][...]
                        )

            pltpu.sync_copy(
                out_v, out_hbm.at[pl.ds(batch0 + g * _BCHUNK, _BCHUNK)]
            )

        start(0, rows0, sem0)

        @pl.loop(0, _NCHUNKS - 2, step=2)
        def _(g):
            start(g + 1, rows1, sem1)
            wait(rows0, sem0)
            emit(g, rows0, out0)
            start(g + 2, rows0, sem0)
            wait(rows1, sem1)
            emit(g + 1, rows1, out1)

        start(_NCHUNKS - 1, rows1, sem1)
        wait(rows0, sem0)
        emit(_NCHUNKS - 2, rows0, out0)
        wait(rows1, sem1)
        emit(_NCHUNKS - 1, rows1, out1)

    return gather_kernel(table128, idx)
